# Initial kernel scaffold; baseline (speedup 1.0000x reference)
import jax
import jax.numpy as jnp
from jax.experimental import pallas as pl


def kernel(x, Wg, We, be):
    raise NotImplementedError("write your pallas kernel here")


# speedup vs baseline: 4.5462x; 4.5462x over previous
import functools

import jax
import jax.numpy as jnp
from jax import lax
from jax.experimental import pallas as pl
from jax.experimental.pallas import tpu as pltpu
from jax.experimental.pallas import tpu_sc as plsc

E = 64
L = 768
LP = L + 128
T = 2048
TILE = 128
NBLK = T // TILE


def _routing_body(x_ref, wg_ref, xw_ref, dest_ref, offs_ref):
    x = x_ref[:]
    wg = wg_ref[:]
    logits = lax.dot_general(x, wg, (((1,), (1,)), ((), ())),
                             preferred_element_type=jnp.float32)
    m = jnp.max(logits, axis=1, keepdims=True)
    sexp = jnp.sum(jnp.exp(logits - m), axis=1, keepdims=True)
    top_w = 1.0 / sexp
    eids = lax.broadcasted_iota(jnp.int32, (T, E), 1)
    expert = jnp.min(jnp.where(logits == m, eids, E), axis=1, keepdims=True)

    xw_ref[:, 0:L] = x * top_w
    xw_ref[:, L:LP] = jnp.broadcast_to(top_w, (T, LP - L))

    r128 = lax.broadcasted_iota(jnp.int32, (TILE, TILE), 0)
    c128 = lax.broadcasted_iota(jnp.int32, (TILE, TILE), 1)
    tri = (c128 < r128).astype(jnp.float32)
    eids128 = lax.broadcasted_iota(jnp.int32, (TILE, E), 1)

    carry = jnp.zeros((1, E), dtype=jnp.float32)
    ranks = []
    for b in range(NBLK):
        oh = (eids128 == expert[b * TILE:(b + 1) * TILE]).astype(jnp.float32)
        within = lax.dot_general(tri, oh, (((1,), (0,)), ((), ())),
                                 preferred_element_type=jnp.float32)
        rank_full = within + carry
        ranks.append(jnp.sum(rank_full * oh, axis=1, keepdims=True))
        carry = carry + jnp.sum(oh, axis=0, keepdims=True)
    counts = carry

    r64 = lax.broadcasted_iota(jnp.int32, (E, E), 0)
    c64 = lax.broadcasted_iota(jnp.int32, (E, E), 1)
    tri_u = (r64 < c64).astype(jnp.float32)
    offs = lax.dot_general(counts, tri_u, (((1,), (0,)), ((), ())),
                           preferred_element_type=jnp.float32)
    offs_ref[:] = offs.astype(jnp.int32)

    for b in range(NBLK):
        oh = (eids128 == expert[b * TILE:(b + 1) * TILE]).astype(jnp.float32)
        off_t = jnp.sum(offs * oh, axis=1, keepdims=True)
        dest_ref[b * TILE:(b + 1) * TILE, :] = (
            (ranks[b] + off_t).astype(jnp.int32))


def _routing(xf, wg):
    return pl.pallas_call(
        _routing_body,
        out_shape=(
            jax.ShapeDtypeStruct((T, LP), jnp.float32),
            jax.ShapeDtypeStruct((T, 1), jnp.int32),
            jax.ShapeDtypeStruct((1, E), jnp.int32),
        ),
    )(xf, wg)


_NC = 2
_NS = 16
_NW = _NC * _NS
_CHUNK = T // _NW


@functools.cache
def _sc_mesh():
    return plsc.VectorSubcoreMesh(core_axis_name="c", subcore_axis_name="s")


def _sc_wid():
    return lax.axis_index("s") * _NC + lax.axis_index("c")


def _scatter_body(xw_hbm, dest_hbm, xs_out, idx_v, rows_v, sem):
    base = _sc_wid() * _CHUNK
    pltpu.sync_copy(dest_hbm.at[pl.ds(base, _CHUNK)], idx_v)
    pltpu.sync_copy(xw_hbm.at[pl.ds(base, _CHUNK)], rows_v)
    pltpu.async_copy(rows_v, xs_out.at[idx_v], sem).wait()


@functools.cache
def _sc_scatter():
    return pl.kernel(
        _scatter_body,
        out_type=jax.ShapeDtypeStruct((T, LP), jnp.float32),
        mesh=_sc_mesh(),
        scratch_types=[
            pltpu.VMEM((_CHUNK,), jnp.int32),
            pltpu.VMEM((_CHUNK, LP), jnp.float32),
            pltpu.SemaphoreType.DMA,
        ],
    )


def _gather_body(y_hbm, dest_hbm, out_hbm, idx_v, rows_v, sem):
    base = _sc_wid() * _CHUNK
    pltpu.sync_copy(dest_hbm.at[pl.ds(base, _CHUNK)], idx_v)
    pltpu.async_copy(y_hbm.at[idx_v], rows_v, sem).wait()
    pltpu.sync_copy(rows_v, out_hbm.at[pl.ds(base, _CHUNK)])


@functools.cache
def _sc_gather():
    return pl.kernel(
        _gather_body,
        out_type=jax.ShapeDtypeStruct((T, L), jnp.float32),
        mesh=_sc_mesh(),
        scratch_types=[
            pltpu.VMEM((_CHUNK,), jnp.int32),
            pltpu.VMEM((_CHUNK, L), jnp.float32),
            pltpu.SemaphoreType.DMA,
        ],
    )


def _expert_body(offs_ref, xs_ref, we_ref, be_ref, out_ref):
    e = pl.program_id(0)

    @pl.when(e == 0)
    def _():
        out_ref[:] = jnp.zeros_like(out_ref)

    lo = offs_ref[e]
    hi = offs_ref[e + 1]
    w = we_ref[0]
    b = be_ref[0]

    def body(t, _):
        rows = xs_ref[pl.ds(t * TILE, TILE), 0:L]
        y = lax.dot_general(rows, w, (((1,), (1,)), ((), ())),
                            preferred_element_type=jnp.float32)
        tw = xs_ref[pl.ds(t * TILE, TILE), L:L + 1]
        y = y + tw * b
        rid = t * TILE + lax.broadcasted_iota(jnp.int32, (TILE, 1), 0)
        mask = (rid >= lo) & (rid < hi)
        cur = out_ref[pl.ds(t * TILE, TILE), :]
        out_ref[pl.ds(t * TILE, TILE), :] = cur + jnp.where(mask, y, 0.0)
        return 0

    lax.fori_loop(lo // TILE, (hi + TILE - 1) // TILE, body, 0)


def _expert_matmuls(offs65, xs, we, be3):
    grid_spec = pltpu.PrefetchScalarGridSpec(
        num_scalar_prefetch=1,
        grid=(E,),
        in_specs=[
            pl.BlockSpec((T, LP), lambda e, offs: (0, 0)),
            pl.BlockSpec((1, L, L), lambda e, offs: (e, 0, 0)),
            pl.BlockSpec((1, 1, L), lambda e, offs: (e, 0, 0)),
        ],
        out_specs=pl.BlockSpec((T, L), lambda e, offs: (0, 0)),
    )
    return pl.pallas_call(
        _expert_body,
        grid_spec=grid_spec,
        out_shape=jax.ShapeDtypeStruct((T, L), jnp.float32),
    )(offs65, xs, we, be3)


def kernel(x, Wg, We, be):
    Bq, Cq, Lq = x.shape
    xf = x.reshape(T, L)
    xw, dest2d, offs = _routing(xf, Wg)
    dest = dest2d.reshape(T)
    offs65 = jnp.concatenate([offs.reshape(E), jnp.full((1,), T, jnp.int32)])
    xs = _sc_scatter()(xw, dest)
    y_sorted = _expert_matmuls(offs65, xs, We, be.reshape(E, 1, L))
    out = _sc_gather()(y_sorted, dest)
    return out.reshape(Bq, Cq, Lq)
